# baseline (speedup 1.0000x reference)
import functools

import jax
import jax.numpy as jnp
from jax.experimental import pallas as pl
from jax.experimental.pallas import tpu as pltpu


def _conv_pool_body(x_ref, w_ref, b_ref, s_ref, i_ref, o_ref, *, H, W, GC, K, GHD, U):
    for u in range(U):
        _conv_pool_one(
            x_ref, w_ref, b_ref, s_ref, i_ref, o_ref,
            u=u, H=H, W=W, GC=GC, K=K, GHD=GHD,
        )


def _conv_pool_one(x_ref, w_ref, b_ref, s_ref, i_ref, o_ref, *, u, H, W, GC, K, GHD):
    HW = H * W
    pad = (K - 1) // 2
    xv = (
        jnp.transpose(x_ref[u], (1, 2, 0))
        .reshape(HW, GC)
        .astype(jnp.bfloat16)
    )
    z = jnp.zeros((pad * W, GC), xv.dtype)
    xp = jnp.concatenate([z, xv, z], axis=0)
    PW = (H + 2 * pad) * W
    w_idx = jax.lax.broadcasted_iota(jnp.int32, (PW, 1), 0) % W
    z1 = jnp.zeros((1, GC), xv.dtype)
    variants = {0: xp}
    for dw in range(-pad, pad + 1):
        if dw == 0:
            continue
        if dw < 0:
            v = jnp.concatenate([jnp.tile(z1, (-dw, 1)), xp[:dw]], axis=0)
        else:
            v = jnp.concatenate([xp[dw:], jnp.tile(z1, (dw, 1))], axis=0)
        valid = (w_idx + dw >= 0) & (w_idx + dw < W)
        variants[dw] = jnp.where(valid, v, jnp.zeros((), v.dtype))
    taps = [
        variants[kw - pad][kh * W : kh * W + HW]
        for kh in range(K)
        for kw in range(K)
    ]
    patches = jnp.concatenate(taps, axis=1)
    a = jnp.dot(patches, w_ref[...], preferred_element_type=jnp.float32)
    a1 = jnp.concatenate([a[1:], a[HW - 1 :]], axis=0)
    m1 = jnp.maximum(a, a1)
    m2 = jnp.maximum(m1[: HW - W], m1[W:])
    mc = jnp.dot(
        s_ref[...], m2.astype(jnp.bfloat16),
        preferred_element_type=jnp.float32,
    )
    o_ref[u] = jnp.maximum(mc + b_ref[...], 0.0).astype(o_ref.dtype)


def _head_body(x_ref, w_ref, b_ref, o_ref):
    o_ref[...] = (
        jnp.dot(x_ref[...], w_ref[...], preferred_element_type=jnp.float32)
        + b_ref[...]
    )


@jax.jit
def kernel(images, conv_w, conv_b, lin_w, lin_b):
    N, C, H, W = images.shape
    HD = conv_w.shape[0]
    K = conv_w.shape[2]
    NCLS = lin_w.shape[0]
    Ho, Wo = H // 2, W // 2
    HWo = Ho * Wo
    G = 4
    U = 4
    NB = N // G
    bf16 = jnp.bfloat16

    x = images.reshape(NB, G * C, H, W)
    eye_gc = jnp.eye(G * C, dtype=bf16)
    wt = jnp.transpose(conv_w, (2, 3, 1, 0)).reshape(K * K, C, HD)
    eye_g = jnp.eye(G, dtype=wt.dtype)
    w4 = jnp.einsum("gh,tcd->tgchd", eye_g, wt)
    w4 = w4.reshape(K * K * G * C, G * HD).astype(bf16)
    b4 = jnp.tile(conv_b, G).reshape(1, G * HD)
    q = jnp.arange(HWo)
    sel = jax.nn.one_hot(
        2 * W * (q // Wo) + 2 * (q % Wo), H * W - W, dtype=bf16
    )

    pooled = pl.pallas_call(
        functools.partial(
            _conv_pool_body, H=H, W=W, GC=G * C, K=K, GHD=G * HD, U=U
        ),
        out_shape=jax.ShapeDtypeStruct((NB, HWo, G * HD), bf16),
        grid=(NB // U,),
        in_specs=[
            pl.BlockSpec((U, G * C, H, W), lambda n: (n, 0, 0, 0)),
            pl.BlockSpec((K * K * G * C, G * HD), lambda n: (0, 0)),
            pl.BlockSpec((1, G * HD), lambda n: (0, 0)),
            pl.BlockSpec((HWo, H * W - W), lambda n: (0, 0)),
            pl.BlockSpec((G * C, G * C), lambda n: (0, 0)),
        ],
        out_specs=pl.BlockSpec((U, HWo, G * HD), lambda n: (n, 0, 0)),
        compiler_params=pltpu.CompilerParams(
            dimension_semantics=("parallel",)
        ),
    )(x, w4, b4, sel, eye_gc)

    feats_flat = pooled.reshape(NB, HWo * G * HD)
    wl5 = lin_w.reshape(NCLS, HD, Ho, Wo).transpose(2, 3, 1, 0)
    wl4 = jnp.einsum(
        "gh,ijdc->ijgdhc", jnp.eye(G, dtype=wl5.dtype), wl5
    ).reshape(HWo * G * HD, G * NCLS).astype(bf16)
    bl4 = jnp.tile(lin_b, G).reshape(1, G * NCLS)

    logits4 = pl.pallas_call(
        _head_body,
        out_shape=jax.ShapeDtypeStruct((NB, G * NCLS), jnp.float32),
    )(feats_flat, wl4, bl4)
    return logits4.reshape(N, NCLS)


# speedup vs baseline: 1.0450x; 1.0450x over previous
import functools

import jax
import jax.numpy as jnp
from jax.experimental import pallas as pl
from jax.experimental.pallas import tpu as pltpu


def _conv_pool_body(x_ref, w_ref, b_ref, s_ref, i_ref, o_ref, *, H, W, GC, K, GHD, U):
    for u in range(U):
        _conv_pool_one(
            x_ref, w_ref, b_ref, s_ref, i_ref, o_ref,
            u=u, H=H, W=W, GC=GC, K=K, GHD=GHD,
        )


def _conv_pool_one(x_ref, w_ref, b_ref, s_ref, i_ref, o_ref, *, u, H, W, GC, K, GHD):
    HW = H * W
    pad = (K - 1) // 2
    xv = (
        jnp.transpose(x_ref[u].astype(jnp.bfloat16), (1, 2, 0))
        .reshape(HW, GC)
    )
    z = jnp.zeros((pad * W, GC), xv.dtype)
    xp = jnp.concatenate([z, xv, z], axis=0)
    PW = (H + 2 * pad) * W
    w_idx = jax.lax.broadcasted_iota(jnp.int32, (PW, 1), 0) % W
    z1 = jnp.zeros((1, GC), xv.dtype)
    variants = {0: xp}
    for dw in range(-pad, pad + 1):
        if dw == 0:
            continue
        if dw < 0:
            v = jnp.concatenate([jnp.tile(z1, (-dw, 1)), xp[:dw]], axis=0)
        else:
            v = jnp.concatenate([xp[dw:], jnp.tile(z1, (dw, 1))], axis=0)
        valid = (w_idx + dw >= 0) & (w_idx + dw < W)
        variants[dw] = jnp.where(valid, v, jnp.zeros((), v.dtype))
    taps = [
        variants[kw - pad][kh * W : kh * W + HW]
        for kh in range(K)
        for kw in range(K)
    ]
    patches = jnp.concatenate(taps, axis=1)
    a = jnp.dot(patches, w_ref[...], preferred_element_type=jnp.float32)
    a1 = jnp.concatenate([a[1:], a[HW - 1 :]], axis=0)
    m1 = jnp.maximum(a, a1)
    m2 = jnp.maximum(m1[: HW - W], m1[W:])
    mc = jnp.dot(
        s_ref[...], m2.astype(jnp.bfloat16),
        preferred_element_type=jnp.float32,
    )
    o_ref[u] = jnp.maximum(mc + b_ref[...], 0.0).astype(o_ref.dtype)


def _head_body(x_ref, w_ref, b_ref, o_ref):
    o_ref[...] = (
        jnp.dot(x_ref[...], w_ref[...], preferred_element_type=jnp.float32)
        + b_ref[...]
    )


@jax.jit
def kernel(images, conv_w, conv_b, lin_w, lin_b):
    N, C, H, W = images.shape
    HD = conv_w.shape[0]
    K = conv_w.shape[2]
    NCLS = lin_w.shape[0]
    Ho, Wo = H // 2, W // 2
    HWo = Ho * Wo
    G = 4
    U = 4
    NB = N // G
    bf16 = jnp.bfloat16

    x = images.reshape(NB, G * C, H, W)
    eye_gc = jnp.eye(G * C, dtype=bf16)
    wt = jnp.transpose(conv_w, (2, 3, 1, 0)).reshape(K * K, C, HD)
    eye_g = jnp.eye(G, dtype=wt.dtype)
    w4 = jnp.einsum("gh,tcd->tgchd", eye_g, wt)
    w4 = w4.reshape(K * K * G * C, G * HD).astype(bf16)
    b4 = jnp.tile(conv_b, G).reshape(1, G * HD)
    q = jnp.arange(HWo)
    sel = jax.nn.one_hot(
        2 * W * (q // Wo) + 2 * (q % Wo), H * W - W, dtype=bf16
    )

    pooled = pl.pallas_call(
        functools.partial(
            _conv_pool_body, H=H, W=W, GC=G * C, K=K, GHD=G * HD, U=U
        ),
        out_shape=jax.ShapeDtypeStruct((NB, HWo, G * HD), bf16),
        grid=(NB // U,),
        in_specs=[
            pl.BlockSpec((U, G * C, H, W), lambda n: (n, 0, 0, 0)),
            pl.BlockSpec((K * K * G * C, G * HD), lambda n: (0, 0)),
            pl.BlockSpec((1, G * HD), lambda n: (0, 0)),
            pl.BlockSpec((HWo, H * W - W), lambda n: (0, 0)),
            pl.BlockSpec((G * C, G * C), lambda n: (0, 0)),
        ],
        out_specs=pl.BlockSpec((U, HWo, G * HD), lambda n: (n, 0, 0)),
        compiler_params=pltpu.CompilerParams(
            dimension_semantics=("parallel",)
        ),
    )(x, w4, b4, sel, eye_gc)

    feats_flat = pooled.reshape(NB, HWo * G * HD)
    wl5 = lin_w.reshape(NCLS, HD, Ho, Wo).transpose(2, 3, 1, 0)
    wl4 = jnp.einsum(
        "gh,ijdc->ijgdhc", jnp.eye(G, dtype=wl5.dtype), wl5
    ).reshape(HWo * G * HD, G * NCLS).astype(bf16)
    bl4 = jnp.tile(lin_b, G).reshape(1, G * NCLS)

    logits4 = pl.pallas_call(
        _head_body,
        out_shape=jax.ShapeDtypeStruct((NB, G * NCLS), jnp.float32),
    )(feats_flat, wl4, bl4)
    return logits4.reshape(N, NCLS)


# speedup vs baseline: 1.0614x; 1.0157x over previous
import functools

import jax
import jax.numpy as jnp
from jax.experimental import pallas as pl
from jax.experimental.pallas import tpu as pltpu


def _conv_pool_body(x_ref, w_ref, b_ref, s_ref, i_ref, o_ref, *, H, W, GC, K, GHD, U):
    for u in range(U):
        _conv_pool_one(
            x_ref, w_ref, b_ref, s_ref, i_ref, o_ref,
            u=u, H=H, W=W, GC=GC, K=K, GHD=GHD,
        )


def _conv_pool_one(x_ref, w_ref, b_ref, s_ref, i_ref, o_ref, *, u, H, W, GC, K, GHD):
    HW = H * W
    pad = (K - 1) // 2
    xv = (
        jnp.transpose(x_ref[u].astype(jnp.bfloat16), (1, 2, 0))
        .reshape(HW, GC)
    )
    z = jnp.zeros((pad * W, GC), xv.dtype)
    xp = jnp.concatenate([z, xv, z], axis=0)
    PW = (H + 2 * pad) * W
    w_idx = jax.lax.broadcasted_iota(jnp.int32, (PW, 1), 0) % W
    z1 = jnp.zeros((1, GC), xv.dtype)
    variants = {0: xp}
    for dw in range(-pad, pad + 1):
        if dw == 0:
            continue
        if dw < 0:
            v = jnp.concatenate([jnp.tile(z1, (-dw, 1)), xp[:dw]], axis=0)
        else:
            v = jnp.concatenate([xp[dw:], jnp.tile(z1, (dw, 1))], axis=0)
        valid = (w_idx + dw >= 0) & (w_idx + dw < W)
        variants[dw] = jnp.where(valid, v, jnp.zeros((), v.dtype))
    taps = [
        variants[kw - pad][kh * W : kh * W + HW]
        for kh in range(K)
        for kw in range(K)
    ]
    patches = jnp.concatenate(taps, axis=1)
    a = jnp.dot(patches, w_ref[...], preferred_element_type=jnp.float32)
    a1 = jnp.concatenate([a[1:], a[HW - 1 :]], axis=0)
    m1 = jnp.maximum(a, a1)
    m2 = jnp.maximum(m1[: HW - W], m1[W:])
    mc = jnp.dot(
        s_ref[...], m2.astype(jnp.bfloat16),
        preferred_element_type=jnp.float32,
    )
    o_ref[u] = jnp.maximum(mc + b_ref[...], 0.0).astype(o_ref.dtype)


def _head_body(x_ref, w_ref, b_ref, o_ref):
    o_ref[...] = (
        jnp.dot(x_ref[...], w_ref[...], preferred_element_type=jnp.float32)
        + b_ref[...]
    )


@jax.jit
def kernel(images, conv_w, conv_b, lin_w, lin_b):
    N, C, H, W = images.shape
    HD = conv_w.shape[0]
    K = conv_w.shape[2]
    NCLS = lin_w.shape[0]
    Ho, Wo = H // 2, W // 2
    HWo = Ho * Wo
    G = 4
    U = 8
    NB = N // G
    bf16 = jnp.bfloat16

    x = images.reshape(NB, G * C, H, W)
    eye_gc = jnp.eye(G * C, dtype=bf16)
    wt = jnp.transpose(conv_w, (2, 3, 1, 0)).reshape(K * K, C, HD)
    eye_g = jnp.eye(G, dtype=wt.dtype)
    w4 = jnp.einsum("gh,tcd->tgchd", eye_g, wt)
    w4 = w4.reshape(K * K * G * C, G * HD).astype(bf16)
    b4 = jnp.tile(conv_b, G).reshape(1, G * HD)
    q = jnp.arange(HWo)
    sel = jax.nn.one_hot(
        2 * W * (q // Wo) + 2 * (q % Wo), H * W - W, dtype=bf16
    )

    pooled = pl.pallas_call(
        functools.partial(
            _conv_pool_body, H=H, W=W, GC=G * C, K=K, GHD=G * HD, U=U
        ),
        out_shape=jax.ShapeDtypeStruct((NB, HWo, G * HD), bf16),
        grid=(NB // U,),
        in_specs=[
            pl.BlockSpec((U, G * C, H, W), lambda n: (n, 0, 0, 0)),
            pl.BlockSpec((K * K * G * C, G * HD), lambda n: (0, 0)),
            pl.BlockSpec((1, G * HD), lambda n: (0, 0)),
            pl.BlockSpec((HWo, H * W - W), lambda n: (0, 0)),
            pl.BlockSpec((G * C, G * C), lambda n: (0, 0)),
        ],
        out_specs=pl.BlockSpec((U, HWo, G * HD), lambda n: (n, 0, 0)),
        compiler_params=pltpu.CompilerParams(
            dimension_semantics=("parallel",)
        ),
    )(x, w4, b4, sel, eye_gc)

    feats_flat = pooled.reshape(NB, HWo * G * HD)
    wl5 = lin_w.reshape(NCLS, HD, Ho, Wo).transpose(2, 3, 1, 0)
    wl4 = jnp.einsum(
        "gh,ijdc->ijgdhc", jnp.eye(G, dtype=wl5.dtype), wl5
    ).reshape(HWo * G * HD, G * NCLS).astype(bf16)
    bl4 = jnp.tile(lin_b, G).reshape(1, G * NCLS)

    logits4 = pl.pallas_call(
        _head_body,
        out_shape=jax.ShapeDtypeStruct((NB, G * NCLS), jnp.float32),
    )(feats_flat, wl4, bl4)
    return logits4.reshape(N, NCLS)


# speedup vs baseline: 1.0629x; 1.0014x over previous
import functools

import jax
import jax.numpy as jnp
from jax.experimental import pallas as pl
from jax.experimental.pallas import tpu as pltpu


def _conv_pool_body(x_ref, w_ref, b_ref, s_ref, o_ref, *, H, W, GC, K, GHD, U):
    for u in range(U):
        _conv_pool_one(
            x_ref, w_ref, b_ref, s_ref, o_ref,
            u=u, H=H, W=W, GC=GC, K=K, GHD=GHD,
        )


def _conv_pool_one(x_ref, w_ref, b_ref, s_ref, o_ref, *, u, H, W, GC, K, GHD):
    HW = H * W
    pad = (K - 1) // 2
    xv = (
        jnp.transpose(x_ref[u].astype(jnp.bfloat16), (1, 2, 0))
        .reshape(HW, GC)
    )
    z = jnp.zeros((pad * W, GC), xv.dtype)
    xp = jnp.concatenate([z, xv, z], axis=0)
    PW = (H + 2 * pad) * W
    w_idx = jax.lax.broadcasted_iota(jnp.int32, (PW, 1), 0) % W
    z1 = jnp.zeros((1, GC), xv.dtype)
    variants = {0: xp}
    for dw in range(-pad, pad + 1):
        if dw == 0:
            continue
        if dw < 0:
            v = jnp.concatenate([jnp.tile(z1, (-dw, 1)), xp[:dw]], axis=0)
        else:
            v = jnp.concatenate([xp[dw:], jnp.tile(z1, (dw, 1))], axis=0)
        valid = (w_idx + dw >= 0) & (w_idx + dw < W)
        variants[dw] = jnp.where(valid, v, jnp.zeros((), v.dtype))
    taps = [
        variants[kw - pad][kh * W : kh * W + HW]
        for kh in range(K)
        for kw in range(K)
    ]
    patches = jnp.concatenate(taps, axis=1)
    a = jnp.dot(patches, w_ref[...], preferred_element_type=jnp.float32)
    a1 = jnp.concatenate([a[1:], a[HW - 1 :]], axis=0)
    m1 = jnp.maximum(a, a1)
    m2 = jnp.maximum(m1[: HW - W], m1[W:])
    mc = jnp.dot(
        s_ref[...], m2.astype(jnp.bfloat16),
        preferred_element_type=jnp.float32,
    )
    o_ref[u] = jnp.maximum(mc + b_ref[...], 0.0).astype(o_ref.dtype)


def _head_body(x_ref, w_ref, b_ref, o_ref):
    o_ref[...] = (
        jnp.dot(x_ref[...], w_ref[...], preferred_element_type=jnp.float32)
        + b_ref[...]
    )


@jax.jit
def kernel(images, conv_w, conv_b, lin_w, lin_b):
    N, C, H, W = images.shape
    HD = conv_w.shape[0]
    K = conv_w.shape[2]
    NCLS = lin_w.shape[0]
    Ho, Wo = H // 2, W // 2
    HWo = Ho * Wo
    G = 4
    NB = N // G
    U = min(8, NB)
    bf16 = jnp.bfloat16

    x = images.reshape(NB, G * C, H, W)
    wt = jnp.transpose(conv_w, (2, 3, 1, 0)).reshape(K * K, C, HD)
    eye_g = jnp.eye(G, dtype=wt.dtype)
    w4 = jnp.einsum("gh,tcd->tgchd", eye_g, wt)
    w4 = w4.reshape(K * K * G * C, G * HD).astype(bf16)
    b4 = jnp.tile(conv_b, G).reshape(1, G * HD)
    q = jnp.arange(HWo)
    sel = jax.nn.one_hot(
        2 * W * (q // Wo) + 2 * (q % Wo), H * W - W, dtype=bf16
    )

    pooled = pl.pallas_call(
        functools.partial(
            _conv_pool_body, H=H, W=W, GC=G * C, K=K, GHD=G * HD, U=U
        ),
        out_shape=jax.ShapeDtypeStruct((NB, HWo, G * HD), bf16),
        grid=(NB // U,),
        in_specs=[
            pl.BlockSpec((U, G * C, H, W), lambda n: (n, 0, 0, 0)),
            pl.BlockSpec((K * K * G * C, G * HD), lambda n: (0, 0)),
            pl.BlockSpec((1, G * HD), lambda n: (0, 0)),
            pl.BlockSpec((HWo, H * W - W), lambda n: (0, 0)),
        ],
        out_specs=pl.BlockSpec((U, HWo, G * HD), lambda n: (n, 0, 0)),
        compiler_params=pltpu.CompilerParams(
            dimension_semantics=("parallel",)
        ),
    )(x, w4, b4, sel)

    feats_flat = pooled.reshape(NB, HWo * G * HD)
    wl5 = lin_w.reshape(NCLS, HD, Ho, Wo).transpose(2, 3, 1, 0)
    wl4 = jnp.einsum(
        "gh,ijdc->ijgdhc", jnp.eye(G, dtype=wl5.dtype), wl5
    ).reshape(HWo * G * HD, G * NCLS).astype(bf16)
    bl4 = jnp.tile(lin_b, G).reshape(1, G * NCLS)

    logits4 = pl.pallas_call(
        _head_body,
        out_shape=jax.ShapeDtypeStruct((NB, G * NCLS), jnp.float32),
    )(feats_flat, wl4, bl4)
    return logits4.reshape(N, NCLS)
